# baseline (speedup 1.0000x reference)
import functools

import jax
import jax.numpy as jnp
from jax import lax
from jax.experimental import pallas as pl
from jax.experimental.pallas import tpu as pltpu
from jax.experimental.pallas import tpu_sc as plsc

N = 10000
E = 320000
NPAD = 10240
CHUNK = 128
NC, NS = 2, 16
CPT_A = 80
CPT_B = 160
ECH = 2560
E_PAD = ECH * CHUNK

_mesh = plsc.VectorSubcoreMesh(
    core_axis_name="c", subcore_axis_name="s", num_cores=NC, num_subcores=NS)


def _deg_body(dst2d, ones128, zeros128, out, accum, dst_v, ones_v):
    c = lax.axis_index("c")
    s = lax.axis_index("s")
    w = c * NS + s
    pltpu.sync_copy(dst2d.at[pl.ds(w * CPT_A, CPT_A)], dst_v)
    pltpu.sync_copy(ones128, ones_v)
    _zero_accum(zeros128, accum, s)
    plsc.subcore_barrier()

    def chunk(j, carry):
        pltpu.sync_copy(ones_v, accum.at[dst_v.at[j]], add=True)
        return carry

    lax.fori_loop(0, CPT_A, chunk, 0)
    plsc.subcore_barrier()
    base = s * (NPAD // NS)
    for half in range(2):
        r0 = base + half * 320
        pltpu.sync_copy(accum.at[pl.ds(r0, 320)], out.at[c, pl.ds(r0, 320)])


_deg_call = pl.kernel(
    _deg_body,
    out_type=jax.ShapeDtypeStruct((NC, NPAD, 128), jnp.float32),
    mesh=_mesh,
    scratch_types=[
        pltpu.VMEM_SHARED((NPAD, 128), jnp.float32),
        pltpu.VMEM((CPT_A, CHUNK), jnp.int32),
        pltpu.VMEM((CHUNK, 128), jnp.float32),
    ],
)


def _zero_accum(zeros_hbm, accum, s):
    base = s * (NPAD // NS)
    pltpu.sync_copy(zeros_hbm, accum.at[pl.ds(base, 320)])
    pltpu.sync_copy(zeros_hbm, accum.at[pl.ds(base + 320, 320)])


IBLK = 16


def _mp_common(gather_src, chunk0, cpt, src_hbm_slice, dst2d, zeros128, out,
               accum, src_v, dst_v, rows0, rows1,
               gsem0, gsem1, ssem0, ssem1, isem0, isem1, c, s):
    base = s * (NPAD // NS)
    nblk = cpt // IBLK
    pltpu.sync_copy(src_hbm_slice.at[pl.ds(chunk0, IBLK)], src_v.at[0])
    pltpu.sync_copy(dst2d.at[pl.ds(chunk0, IBLK)], dst_v.at[0])
    _zero_accum(zeros128, accum, s)
    plsc.subcore_barrier()
    rows = (rows0, rows1)
    gsem = (gsem0, gsem1)
    ssem = (ssem0, ssem1)

    def block(b, carry):
        p = lax.rem(b, 2)
        pn = lax.rem(b + 1, 2)
        nc0 = chunk0 + (b + 1) * IBLK

        @pl.when(b + 1 < nblk)
        def _():
            pltpu.async_copy(src_hbm_slice.at[pl.ds(nc0, IBLK)],
                             src_v.at[pn], isem0)
            pltpu.async_copy(dst2d.at[pl.ds(nc0, IBLK)], dst_v.at[pn], isem1)

        sv = src_v.at[p]
        dv = dst_v.at[p]
        gd = [None, None]
        sd = [None, None]
        gd[0] = pltpu.async_copy(gather_src.at[sv.at[0]], rows[0], gsem[0])
        for j in range(IBLK):
            a = j % 2
            gd[a].wait()
            sd[a] = pltpu.async_copy(rows[a], accum.at[dv.at[j]], ssem[a],
                                     add=True)
            if j + 1 < IBLK:
                nb = (j + 1) % 2
                if sd[nb] is not None:
                    sd[nb].wait()
                gd[nb] = pltpu.async_copy(gather_src.at[sv.at[j + 1]],
                                          rows[nb], gsem[nb])
        sd[(IBLK - 2) % 2].wait()
        sd[(IBLK - 1) % 2].wait()

        @pl.when(b + 1 < nblk)
        def _():
            pltpu.make_async_copy(src_hbm_slice.at[pl.ds(nc0, IBLK)],
                                  src_v.at[pn], isem0).wait()
            pltpu.make_async_copy(dst2d.at[pl.ds(nc0, IBLK)],
                                  dst_v.at[pn], isem1).wait()
        return carry

    lax.fori_loop(0, nblk, block, 0)
    plsc.subcore_barrier()
    for half in range(2):
        r0 = base + half * 320
        pltpu.sync_copy(accum.at[pl.ds(r0, 320)], out.at[c, pl.ds(r0, 320)])


def _mp128_body(hp, src2d, dst2d, zeros128, out, accum, src_v, dst_v,
                rows0, rows1, gsem0, gsem1, ssem0, ssem1, isem0, isem1):
    c = lax.axis_index("c")
    s = lax.axis_index("s")
    w = c * NS + s
    _mp_common(hp, w * CPT_A, CPT_A, src2d, dst2d, zeros128, out,
               accum, src_v, dst_v, rows0, rows1,
               gsem0, gsem1, ssem0, ssem1, isem0, isem1, c, s)


_mp_scratch = [
    pltpu.VMEM_SHARED((NPAD, 128), jnp.float32),
    pltpu.VMEM((2, IBLK, CHUNK), jnp.int32),
    pltpu.VMEM((2, IBLK, CHUNK), jnp.int32),
    pltpu.VMEM((CHUNK, 128), jnp.float32),
    pltpu.VMEM((CHUNK, 128), jnp.float32),
    pltpu.SemaphoreType.DMA,
    pltpu.SemaphoreType.DMA,
    pltpu.SemaphoreType.DMA,
    pltpu.SemaphoreType.DMA,
    pltpu.SemaphoreType.DMA,
    pltpu.SemaphoreType.DMA,
]

_mp128_call = pl.kernel(
    _mp128_body,
    out_type=jax.ShapeDtypeStruct((NC, NPAD, 128), jnp.float32),
    mesh=_mesh,
    scratch_types=_mp_scratch,
)


def _mp256_body(hpf, src2, dst2d, zeros128, out, accum, src_v, dst_v,
                rows0, rows1, gsem0, gsem1, ssem0, ssem1, isem0, isem1):
    c = lax.axis_index("c")
    s = lax.axis_index("s")
    _mp_common(hpf, s * CPT_B, CPT_B, src2.at[c], dst2d, zeros128, out,
               accum, src_v, dst_v, rows0, rows1,
               gsem0, gsem1, ssem0, ssem1, isem0, isem1, c, s)


_mp256_call = pl.kernel(
    _mp256_body,
    out_type=jax.ShapeDtypeStruct((NC, NPAD, 128), jnp.float32),
    mesh=_mesh,
    scratch_types=_mp_scratch,
)


BN = 1000

_DOT = functools.partial(jnp.dot, preferred_element_type=jnp.float32)


def _ln(t, g, b):
    m = jnp.mean(t, axis=-1, keepdims=True)
    v = jnp.mean((t - m) ** 2, axis=-1, keepdims=True)
    return (t - m) * lax.rsqrt(v + 1e-5) * g + b


def _elu(t):
    return jnp.where(t > 0, t, jnp.exp(jnp.minimum(t, 0.0)) - 1.0)


def _tc0_body(x_ref, w_ref, hp_ref):
    hp_ref[...] = _DOT(x_ref[...], w_ref[...])


def _tc0(x, W1):
    return pl.pallas_call(
        _tc0_body,
        grid=(N // BN,),
        in_specs=[
            pl.BlockSpec((BN, 128), lambda i: (i, 0)),
            pl.BlockSpec((128, 128), lambda i: (0, 0)),
        ],
        out_specs=pl.BlockSpec((BN, 128), lambda i: (i, 0)),
        out_shape=jax.ShapeDtypeStruct((N, 128), jnp.float32),
    )(x, W1)


def _tc1_body(hpraw_ref, degp_ref, hp_ref, dinv_ref):
    deg = degp_ref[0][:, 0:1] + degp_ref[1][:, 0:1] + 1.0
    dinv = lax.rsqrt(deg)
    hp_ref[...] = hpraw_ref[...] * dinv
    dinv_ref[...] = dinv


def _tc1(hpraw, degp):
    return pl.pallas_call(
        _tc1_body,
        grid=(N // BN,),
        in_specs=[
            pl.BlockSpec((BN, 128), lambda i: (i, 0)),
            pl.BlockSpec((NC, BN, 128), lambda i: (0, i, 0)),
        ],
        out_specs=[
            pl.BlockSpec((BN, 128), lambda i: (i, 0)),
            pl.BlockSpec((BN, 1), lambda i: (i, 0)),
        ],
        out_shape=[
            jax.ShapeDtypeStruct((N, 128), jnp.float32),
            jax.ShapeDtypeStruct((N, 1), jnp.float32),
        ],
    )(hpraw, degp)


def _tc2_body(p_ref, hp_ref, dinv_ref, b_ref, g_ref, be_ref, w_ref, o_ref):
    agg = p_ref[0] + p_ref[1] + hp_ref[...]
    t = agg * dinv_ref[...] + b_ref[...]
    t = _elu(_ln(t, g_ref[...], be_ref[...]))
    h2 = _DOT(t, w_ref[...]) * dinv_ref[...]
    o_ref[0] = h2[:, :128]
    o_ref[1] = h2[:, 128:]


def _tc2(P1, h1p, dinv, b1, g1, be1, W2):
    return pl.pallas_call(
        _tc2_body,
        grid=(N // BN,),
        in_specs=[
            pl.BlockSpec((NC, BN, 128), lambda i: (0, i, 0)),
            pl.BlockSpec((BN, 128), lambda i: (i, 0)),
            pl.BlockSpec((BN, 1), lambda i: (i, 0)),
            pl.BlockSpec((1, 128), lambda i: (0, 0)),
            pl.BlockSpec((1, 128), lambda i: (0, 0)),
            pl.BlockSpec((1, 128), lambda i: (0, 0)),
            pl.BlockSpec((128, 256), lambda i: (0, 0)),
        ],
        out_specs=pl.BlockSpec((NC, BN, 128), lambda i: (0, i, 0)),
        out_shape=jax.ShapeDtypeStruct((NC, N, 128), jnp.float32),
    )(P1, h1p, dinv, b1, g1, be1, W2)


def _tc3_body(p_ref, hp_ref, dinv_ref, b_ref, g_ref, be_ref, w_ref, o_ref):
    agg = jnp.concatenate([p_ref[0], p_ref[1]], axis=-1)
    hpc = jnp.concatenate([hp_ref[0], hp_ref[1]], axis=-1)
    t = (agg + hpc) * dinv_ref[...] + b_ref[...]
    t = _elu(_ln(t, g_ref[...], be_ref[...]))
    o_ref[...] = _DOT(t, w_ref[...]) * dinv_ref[...]


def _tc3(P2, h2p, dinv, b2, g2, be2, W3):
    return pl.pallas_call(
        _tc3_body,
        grid=(N // BN,),
        in_specs=[
            pl.BlockSpec((NC, BN, 128), lambda i: (0, i, 0)),
            pl.BlockSpec((NC, BN, 128), lambda i: (0, i, 0)),
            pl.BlockSpec((BN, 1), lambda i: (i, 0)),
            pl.BlockSpec((1, 256), lambda i: (0, 0)),
            pl.BlockSpec((1, 256), lambda i: (0, 0)),
            pl.BlockSpec((1, 256), lambda i: (0, 0)),
            pl.BlockSpec((256, 128), lambda i: (0, 0)),
        ],
        out_specs=pl.BlockSpec((BN, 128), lambda i: (i, 0)),
        out_shape=jax.ShapeDtypeStruct((N, 128), jnp.float32),
    )(P2, h2p, dinv, b2, g2, be2, W3)


def _tc4_body(p_ref, hp_ref, dinv_ref, b_ref, g_ref, be_ref,
              w1_ref, lb1_ref, g4_ref, be4_ref, w2_ref, lb2_ref, o_ref):
    agg = p_ref[0] + p_ref[1] + hp_ref[...]
    t = agg * dinv_ref[...] + b_ref[...]
    t = _elu(_ln(t, g_ref[...], be_ref[...]))
    m = _DOT(t, w1_ref[...]) + lb1_ref[...]
    m = _elu(_ln(m, g4_ref[...], be4_ref[...]))
    o_ref[...] = _DOT(m, w2_ref[...]) + lb2_ref[...]


def _tc4(P3, h3p, dinv, b3, g3, be3, lW1, lb1, g4, be4, lW2, lb2):
    return pl.pallas_call(
        _tc4_body,
        grid=(N // BN,),
        in_specs=[
            pl.BlockSpec((NC, BN, 128), lambda i: (0, i, 0)),
            pl.BlockSpec((BN, 128), lambda i: (i, 0)),
            pl.BlockSpec((BN, 1), lambda i: (i, 0)),
            pl.BlockSpec((1, 128), lambda i: (0, 0)),
            pl.BlockSpec((1, 128), lambda i: (0, 0)),
            pl.BlockSpec((1, 128), lambda i: (0, 0)),
            pl.BlockSpec((128, 64), lambda i: (0, 0)),
            pl.BlockSpec((1, 64), lambda i: (0, 0)),
            pl.BlockSpec((1, 64), lambda i: (0, 0)),
            pl.BlockSpec((1, 64), lambda i: (0, 0)),
            pl.BlockSpec((64, 32), lambda i: (0, 0)),
            pl.BlockSpec((1, 32), lambda i: (0, 0)),
        ],
        out_specs=pl.BlockSpec((BN, 32), lambda i: (i, 0)),
        out_shape=jax.ShapeDtypeStruct((N, 32), jnp.float32),
    )(P3, h3p, dinv, b3, g3, be3, lW1, lb1, g4, be4, lW2, lb2)


def kernel(x, edge_index, W1, b1, g1, be1, W2, b2, g2, be2, W3, b3, g3, be3,
           lW1, lb1, g4, be4, lW2, lb2):
    ei = edge_index.astype(jnp.int32)
    pad = E_PAD - E
    pad_i = jnp.arange(pad, dtype=jnp.int32)
    src_p = jnp.concatenate([ei[0], pad_i % N])
    dst_p = jnp.concatenate([ei[1], N + pad_i % (NPAD - N)])
    src2d = src_p.reshape(ECH, CHUNK)
    dst2d = dst_p.reshape(ECH, CHUNK)
    src2 = jnp.stack([src2d, src2d + N])
    zeros128 = jnp.zeros((320, 128), jnp.float32)
    ones128 = jnp.ones((CHUNK, 128), jnp.float32)

    r = lambda a: a.reshape(1, -1)

    degp = _deg_call(dst2d, ones128, zeros128)
    hp1raw = _tc0(x, W1)
    h1p, dinv = _tc1(hp1raw, degp)
    P1 = _mp128_call(h1p, src2d, dst2d, zeros128)
    h2p = _tc2(P1, h1p, dinv, r(b1), r(g1), r(be1), W2)
    P2 = _mp256_call(h2p.reshape(2 * N, 128), src2, dst2d, zeros128)
    h3p = _tc3(P2, h2p, dinv, r(b2), r(g2), r(be2), W3)
    P3 = _mp128_call(h3p, src2d, dst2d, zeros128)
    return _tc4(P3, h3p, dinv, r(b3), r(g3), r(be3),
                lW1, r(lb1), r(g4), r(be4), lW2, r(lb2))


# speedup vs baseline: 1.0043x; 1.0043x over previous
import functools

import jax
import jax.numpy as jnp
from jax import lax
from jax.experimental import pallas as pl
from jax.experimental.pallas import tpu as pltpu
from jax.experimental.pallas import tpu_sc as plsc

N = 10000
E = 320000
NPAD = 10240
CHUNK = 128
NC, NS = 2, 16
CPT_A = 80
CPT_B = 160
ECH = 2560
E_PAD = ECH * CHUNK

_mesh = plsc.VectorSubcoreMesh(
    core_axis_name="c", subcore_axis_name="s", num_cores=NC, num_subcores=NS)


def _deg_body(dst2d, ones128, zeros128, out, accum, dst_v, ones_v):
    c = lax.axis_index("c")
    s = lax.axis_index("s")
    w = c * NS + s
    pltpu.sync_copy(dst2d.at[pl.ds(w * CPT_A, CPT_A)], dst_v)
    pltpu.sync_copy(ones128, ones_v)
    _zero_accum(zeros128, accum, s)
    plsc.subcore_barrier()

    def chunk(j, carry):
        pltpu.sync_copy(ones_v, accum.at[dst_v.at[j]], add=True)
        return carry

    lax.fori_loop(0, CPT_A, chunk, 0)
    plsc.subcore_barrier()
    base = s * (NPAD // NS)
    for half in range(2):
        r0 = base + half * 320
        pltpu.sync_copy(accum.at[pl.ds(r0, 320)], out.at[c, pl.ds(r0, 320)])


_deg_call = pl.kernel(
    _deg_body,
    out_type=jax.ShapeDtypeStruct((NC, NPAD, 128), jnp.float32),
    mesh=_mesh,
    scratch_types=[
        pltpu.VMEM_SHARED((NPAD, 128), jnp.float32),
        pltpu.VMEM((CPT_A, CHUNK), jnp.int32),
        pltpu.VMEM((CHUNK, 128), jnp.float32),
    ],
)


def _zero_accum(zeros_hbm, accum, s):
    base = s * (NPAD // NS)
    pltpu.sync_copy(zeros_hbm, accum.at[pl.ds(base, 320)])
    pltpu.sync_copy(zeros_hbm, accum.at[pl.ds(base + 320, 320)])


IBLK = 16


def _mp_common(gather_src, chunk0, cpt, src_hbm_slice, dst2d, zeros128, out,
               accum, src_v, dst_v, rows0, rows1,
               gsem0, gsem1, ssem0, ssem1, isem0, isem1, c, s):
    base = s * (NPAD // NS)
    nblk = cpt // IBLK
    pltpu.sync_copy(src_hbm_slice.at[pl.ds(chunk0, IBLK)], src_v.at[0])
    pltpu.sync_copy(dst2d.at[pl.ds(chunk0, IBLK)], dst_v.at[0])
    _zero_accum(zeros128, accum, s)
    plsc.subcore_barrier()
    rows = (rows0, rows1)
    gsem = (gsem0, gsem1)
    ssem = (ssem0, ssem1)

    def block(b, carry):
        p = lax.rem(b, 2)
        pn = lax.rem(b + 1, 2)
        nc0 = chunk0 + (b + 1) * IBLK

        @pl.when(b + 1 < nblk)
        def _():
            pltpu.async_copy(src_hbm_slice.at[pl.ds(nc0, IBLK)],
                             src_v.at[pn], isem0)
            pltpu.async_copy(dst2d.at[pl.ds(nc0, IBLK)], dst_v.at[pn], isem1)

        sv = src_v.at[p]
        dv = dst_v.at[p]
        gd = [None, None]
        sd = [None, None]
        gd[0] = pltpu.async_copy(gather_src.at[sv.at[0]], rows[0], gsem[0])
        for j in range(IBLK):
            a = j % 2
            gd[a].wait()
            sd[a] = pltpu.async_copy(rows[a], accum.at[dv.at[j]], ssem[a],
                                     add=True)
            if j + 1 < IBLK:
                nb = (j + 1) % 2
                if sd[nb] is not None:
                    sd[nb].wait()
                gd[nb] = pltpu.async_copy(gather_src.at[sv.at[j + 1]],
                                          rows[nb], gsem[nb])
        sd[(IBLK - 2) % 2].wait()
        sd[(IBLK - 1) % 2].wait()

        @pl.when(b + 1 < nblk)
        def _():
            pltpu.make_async_copy(src_hbm_slice.at[pl.ds(nc0, IBLK)],
                                  src_v.at[pn], isem0).wait()
            pltpu.make_async_copy(dst2d.at[pl.ds(nc0, IBLK)],
                                  dst_v.at[pn], isem1).wait()
        return carry

    lax.fori_loop(0, nblk, block, 0)
    plsc.subcore_barrier()
    for half in range(2):
        r0 = base + half * 320
        pltpu.sync_copy(accum.at[pl.ds(r0, 320)], out.at[c, pl.ds(r0, 320)])


def _mp128_body(hp, src2d, dst2d, zeros128, out, accum, src_v, dst_v,
                rows0, rows1, gsem0, gsem1, ssem0, ssem1, isem0, isem1):
    c = lax.axis_index("c")
    s = lax.axis_index("s")
    w = c * NS + s
    _mp_common(hp, w * CPT_A, CPT_A, src2d, dst2d, zeros128, out,
               accum, src_v, dst_v, rows0, rows1,
               gsem0, gsem1, ssem0, ssem1, isem0, isem1, c, s)


_mp_scratch = [
    pltpu.VMEM_SHARED((NPAD, 128), jnp.float32),
    pltpu.VMEM((2, IBLK, CHUNK), jnp.int32),
    pltpu.VMEM((2, IBLK, CHUNK), jnp.int32),
    pltpu.VMEM((CHUNK, 128), jnp.float32),
    pltpu.VMEM((CHUNK, 128), jnp.float32),
    pltpu.SemaphoreType.DMA,
    pltpu.SemaphoreType.DMA,
    pltpu.SemaphoreType.DMA,
    pltpu.SemaphoreType.DMA,
    pltpu.SemaphoreType.DMA,
    pltpu.SemaphoreType.DMA,
]

_mp128_call = pl.kernel(
    _mp128_body,
    out_type=jax.ShapeDtypeStruct((NC, NPAD, 128), jnp.float32),
    mesh=_mesh,
    scratch_types=_mp_scratch,
)


def _mp256_body(hpf, src2, dst2d, zeros128, out, accum, src_v, dst_v,
                rows0, rows1, gsem0, gsem1, ssem0, ssem1, isem0, isem1):
    c = lax.axis_index("c")
    s = lax.axis_index("s")
    _mp_common(hpf, s * CPT_B, CPT_B, src2.at[c], dst2d, zeros128, out,
               accum, src_v, dst_v, rows0, rows1,
               gsem0, gsem1, ssem0, ssem1, isem0, isem1, c, s)


_mp256_call = pl.kernel(
    _mp256_body,
    out_type=jax.ShapeDtypeStruct((NC, NPAD, 128), jnp.float32),
    mesh=_mesh,
    scratch_types=_mp_scratch,
)


BN = 1000

_DOT = functools.partial(jnp.dot, preferred_element_type=jnp.float32)


def _ln(t, g, b):
    m = jnp.mean(t, axis=-1, keepdims=True)
    v = jnp.mean((t - m) ** 2, axis=-1, keepdims=True)
    return (t - m) * lax.rsqrt(v + 1e-5) * g + b


def _elu(t):
    return jnp.where(t > 0, t, jnp.exp(jnp.minimum(t, 0.0)) - 1.0)


def _tc1_body(x_ref, w_ref, degp_ref, hp_ref, dinv_ref):
    deg = degp_ref[0][:, 0:1] + degp_ref[1][:, 0:1] + 1.0
    dinv = lax.rsqrt(deg)
    hp_ref[...] = _DOT(x_ref[...], w_ref[...]) * dinv
    dinv_ref[...] = dinv


def _tc1(x, W1, degp):
    return pl.pallas_call(
        _tc1_body,
        grid=(N // BN,),
        in_specs=[
            pl.BlockSpec((BN, 128), lambda i: (i, 0)),
            pl.BlockSpec((128, 128), lambda i: (0, 0)),
            pl.BlockSpec((NC, BN, 128), lambda i: (0, i, 0)),
        ],
        out_specs=[
            pl.BlockSpec((BN, 128), lambda i: (i, 0)),
            pl.BlockSpec((BN, 1), lambda i: (i, 0)),
        ],
        out_shape=[
            jax.ShapeDtypeStruct((N, 128), jnp.float32),
            jax.ShapeDtypeStruct((N, 1), jnp.float32),
        ],
    )(x, W1, degp)


EB = ECH // 10


def _prep_body(ei_ref, src_ref, dst_ref, src2_ref):
    b = pl.program_id(0)
    row = lax.broadcasted_iota(jnp.int32, (EB, CHUNK), 0) + b * EB
    lane = lax.broadcasted_iota(jnp.int32, (EB, CHUNK), 1)
    flat = row * CHUNK + lane
    pad_i = flat - E
    is_pad = flat >= E
    src = jnp.where(is_pad, pad_i % N, ei_ref[0])
    dst = jnp.where(is_pad, N + pad_i % (NPAD - N), ei_ref[1])
    src_ref[...] = src
    dst_ref[...] = dst
    src2_ref[0] = src
    src2_ref[1] = src + N


def _prep(ei):
    return pl.pallas_call(
        _prep_body,
        grid=(10,),
        in_specs=[pl.BlockSpec((2, EB, CHUNK), lambda i: (0, i, 0))],
        out_specs=[
            pl.BlockSpec((EB, CHUNK), lambda i: (i, 0)),
            pl.BlockSpec((EB, CHUNK), lambda i: (i, 0)),
            pl.BlockSpec((2, EB, CHUNK), lambda i: (0, i, 0)),
        ],
        out_shape=[
            jax.ShapeDtypeStruct((ECH, CHUNK), jnp.int32),
            jax.ShapeDtypeStruct((ECH, CHUNK), jnp.int32),
            jax.ShapeDtypeStruct((NC, ECH, CHUNK), jnp.int32),
        ],
    )(ei)


def _tc2_body(p_ref, hp_ref, dinv_ref, b_ref, g_ref, be_ref, w_ref, o_ref):
    agg = p_ref[0] + p_ref[1] + hp_ref[...]
    t = agg * dinv_ref[...] + b_ref[...]
    t = _elu(_ln(t, g_ref[...], be_ref[...]))
    h2 = _DOT(t, w_ref[...]) * dinv_ref[...]
    o_ref[0] = h2[:, :128]
    o_ref[1] = h2[:, 128:]


def _tc2(P1, h1p, dinv, b1, g1, be1, W2):
    return pl.pallas_call(
        _tc2_body,
        grid=(N // BN,),
        in_specs=[
            pl.BlockSpec((NC, BN, 128), lambda i: (0, i, 0)),
            pl.BlockSpec((BN, 128), lambda i: (i, 0)),
            pl.BlockSpec((BN, 1), lambda i: (i, 0)),
            pl.BlockSpec((1, 128), lambda i: (0, 0)),
            pl.BlockSpec((1, 128), lambda i: (0, 0)),
            pl.BlockSpec((1, 128), lambda i: (0, 0)),
            pl.BlockSpec((128, 256), lambda i: (0, 0)),
        ],
        out_specs=pl.BlockSpec((NC, BN, 128), lambda i: (0, i, 0)),
        out_shape=jax.ShapeDtypeStruct((NC, N, 128), jnp.float32),
    )(P1, h1p, dinv, b1, g1, be1, W2)


def _tc3_body(p_ref, hp_ref, dinv_ref, b_ref, g_ref, be_ref, w_ref, o_ref):
    agg = jnp.concatenate([p_ref[0], p_ref[1]], axis=-1)
    hpc = jnp.concatenate([hp_ref[0], hp_ref[1]], axis=-1)
    t = (agg + hpc) * dinv_ref[...] + b_ref[...]
    t = _elu(_ln(t, g_ref[...], be_ref[...]))
    o_ref[...] = _DOT(t, w_ref[...]) * dinv_ref[...]


def _tc3(P2, h2p, dinv, b2, g2, be2, W3):
    return pl.pallas_call(
        _tc3_body,
        grid=(N // BN,),
        in_specs=[
            pl.BlockSpec((NC, BN, 128), lambda i: (0, i, 0)),
            pl.BlockSpec((NC, BN, 128), lambda i: (0, i, 0)),
            pl.BlockSpec((BN, 1), lambda i: (i, 0)),
            pl.BlockSpec((1, 256), lambda i: (0, 0)),
            pl.BlockSpec((1, 256), lambda i: (0, 0)),
            pl.BlockSpec((1, 256), lambda i: (0, 0)),
            pl.BlockSpec((256, 128), lambda i: (0, 0)),
        ],
        out_specs=pl.BlockSpec((BN, 128), lambda i: (i, 0)),
        out_shape=jax.ShapeDtypeStruct((N, 128), jnp.float32),
    )(P2, h2p, dinv, b2, g2, be2, W3)


def _tc4_body(p_ref, hp_ref, dinv_ref, b_ref, g_ref, be_ref,
              w1_ref, lb1_ref, g4_ref, be4_ref, w2_ref, lb2_ref, o_ref):
    agg = p_ref[0] + p_ref[1] + hp_ref[...]
    t = agg * dinv_ref[...] + b_ref[...]
    t = _elu(_ln(t, g_ref[...], be_ref[...]))
    m = _DOT(t, w1_ref[...]) + lb1_ref[...]
    m = _elu(_ln(m, g4_ref[...], be4_ref[...]))
    o_ref[...] = _DOT(m, w2_ref[...]) + lb2_ref[...]


def _tc4(P3, h3p, dinv, b3, g3, be3, lW1, lb1, g4, be4, lW2, lb2):
    return pl.pallas_call(
        _tc4_body,
        grid=(N // BN,),
        in_specs=[
            pl.BlockSpec((NC, BN, 128), lambda i: (0, i, 0)),
            pl.BlockSpec((BN, 128), lambda i: (i, 0)),
            pl.BlockSpec((BN, 1), lambda i: (i, 0)),
            pl.BlockSpec((1, 128), lambda i: (0, 0)),
            pl.BlockSpec((1, 128), lambda i: (0, 0)),
            pl.BlockSpec((1, 128), lambda i: (0, 0)),
            pl.BlockSpec((128, 64), lambda i: (0, 0)),
            pl.BlockSpec((1, 64), lambda i: (0, 0)),
            pl.BlockSpec((1, 64), lambda i: (0, 0)),
            pl.BlockSpec((1, 64), lambda i: (0, 0)),
            pl.BlockSpec((64, 32), lambda i: (0, 0)),
            pl.BlockSpec((1, 32), lambda i: (0, 0)),
        ],
        out_specs=pl.BlockSpec((BN, 32), lambda i: (i, 0)),
        out_shape=jax.ShapeDtypeStruct((N, 32), jnp.float32),
    )(P3, h3p, dinv, b3, g3, be3, lW1, lb1, g4, be4, lW2, lb2)


def kernel(x, edge_index, W1, b1, g1, be1, W2, b2, g2, be2, W3, b3, g3, be3,
           lW1, lb1, g4, be4, lW2, lb2):
    ei = edge_index.astype(jnp.int32).reshape(2, E // CHUNK, CHUNK)
    src2d, dst2d, src2 = _prep(ei)
    zeros128 = jnp.zeros((320, 128), jnp.float32)
    ones128 = jnp.ones((CHUNK, 128), jnp.float32)

    r = lambda a: a.reshape(1, -1)

    degp = _deg_call(dst2d, ones128, zeros128)
    h1p, dinv = _tc1(x, W1, degp)
    P1 = _mp128_call(h1p, src2d, dst2d, zeros128)
    h2p = _tc2(P1, h1p, dinv, r(b1), r(g1), r(be1), W2)
    P2 = _mp256_call(h2p.reshape(2 * N, 128), src2, dst2d, zeros128)
    h3p = _tc3(P2, h2p, dinv, r(b2), r(g2), r(be2), W3)
    P3 = _mp128_call(h3p, src2d, dst2d, zeros128)
    return _tc4(P3, h3p, dinv, r(b3), r(g3), r(be3),
                lW1, r(lb1), r(g4), r(be4), lW2, r(lb2))
